# Initial kernel scaffold; baseline (speedup 1.0000x reference)
import jax
import jax.numpy as jnp
from jax.experimental import pallas as pl


def kernel(feats, xyz0, sxyz0, sfeats0, xyz1, sxyz1, sfeats1, xyz2, sxyz2, sfeats2, xyz3, sxyz3, sfeats3, u0_ln1_g, u0_ln1_b, u0_w1, u0_b1, u0_ln2_g, u0_ln2_b, u0_w2, u0_b2, u1_ln1_g, u1_ln1_b, u1_w1, u1_b1, u1_ln2_g, u1_ln2_b, u1_w2, u1_b2, u2_ln1_g, u2_ln1_b, u2_w1, u2_b1, u2_ln2_g, u2_ln2_b, u2_w2, u2_b2, u3_ln1_g, u3_ln1_b, u3_w1, u3_b1, u3_ln2_g, u3_ln2_b, u3_w2, u3_b2, conf_w1, conf_b1, conf_bn_g, conf_bn_b, conf_w2, conf_b2):
    raise NotImplementedError("write your pallas kernel here")


# speedup vs baseline: 4.4801x; 4.4801x over previous
import functools

import jax
import jax.numpy as jnp
from jax.experimental import pallas as pl

_HI = jax.lax.Precision.HIGHEST


def _ln(x, g, b):
    mu = jnp.mean(x, axis=-1, keepdims=True)
    var = jnp.mean((x - mu) ** 2, axis=-1, keepdims=True)
    return (x - mu) / jnp.sqrt(var + 1e-5) * g + b


def _mm(a, b):
    return jax.lax.dot_general(a, b, (((1,), (0,)), ((), ())),
                               precision=_HI, preferred_element_type=jnp.float32)


def _knn_mix(sxyz, xyzT, f2):
    d = None
    for c in range(3):
        diff = sxyz[:, c:c + 1] - xyzT[c:c + 1, :]
        sq = diff * diff
        d = sq if d is None else d + sq
    m_count = xyzT.shape[1]
    iota = jax.lax.broadcasted_iota(jnp.int32, d.shape, 1)
    remaining = d
    sel_w = jnp.zeros_like(d)
    wsum = None
    for r in range(3):
        mval = jnp.min(remaining, axis=1, keepdims=True)
        eq = remaining == mval
        idx = jnp.min(jnp.where(eq, iota, m_count), axis=1, keepdims=True)
        sel = iota == idx
        dist = jnp.sqrt(jnp.maximum(mval, 1e-10))
        w = 1.0 / (dist + 1e-8)
        wsum = w if r == 0 else wsum + w
        sel_w = sel_w + jnp.where(sel, w, 0.0)
        remaining = jnp.where(sel, jnp.inf, remaining)
    return _mm(sel_w / wsum, f2)


def _stages012_body(feats, xyz0T, sxyz0, sfeats0, xyz1T, sxyz1, sfeats1,
                    xyz2T, sxyz2, sfeats2,
                    u0g1, u0b1, u0w1, u0bb1, u0g2, u0b2, u0w2, u0bb2,
                    u1g1, u1b1, u1w1, u1bb1, u1g2, u1b2, u1w2, u1bb2,
                    u2g1, u2b1, u2w1, u2bb1, u2g2, u2b2, u2w2, u2bb2,
                    u3g2, u3b2, u3w2, u3bb2, f2_3_out):
    f = feats[...]
    params = (
        (xyz0T, sxyz0, sfeats0, u0g1, u0b1, u0w1, u0bb1, u0g2, u0b2, u0w2, u0bb2),
        (xyz1T, sxyz1, sfeats1, u1g1, u1b1, u1w1, u1bb1, u1g2, u1b2, u1w2, u1bb2),
        (xyz2T, sxyz2, sfeats2, u2g1, u2b1, u2w1, u2bb1, u2g2, u2b2, u2w2, u2bb2),
    )
    for (xyzT, sxyz, sfeats, g1, b1, w1, bb1, g2, b2, w2, bb2) in params:
        a = _mm(_ln(sfeats[...], g1[...], b1[...]), w1[...]) + bb1[...]
        f2 = _mm(_ln(f, g2[...], b2[...]), w2[...]) + bb2[...]
        f = a + _knn_mix(sxyz[...], xyzT[...], f2)
    f2_3_out[...] = _mm(_ln(f, u3g2[...], u3b2[...]), u3w2[...]) + u3bb2[...]


def _stage3_body(sxyz3, xyz3T, f2_3, sfeats3, g1, b1, w1, bb1,
                 cw1, cb1, h_out):
    a = _mm(_ln(sfeats3[...], g1[...], b1[...]), w1[...]) + bb1[...]
    f = a + _knn_mix(sxyz3[...], xyz3T[...], f2_3[...])
    h_out[...] = _mm(f, cw1[...]) + cb1[...]


def _conf_body(h_ref, bn_g, bn_b, w2T, b2, conf_out):
    h = h_ref[...]
    mu = jnp.mean(h, axis=0, keepdims=True)
    var = jnp.mean((h - mu) ** 2, axis=0, keepdims=True)
    h = (h - mu) / jnp.sqrt(var + 1e-5) * bn_g[...] + bn_b[...]
    h = jnp.maximum(h, 0.0)
    conf_out[...] = jnp.sum(h * w2T[...], axis=1, keepdims=True) + b2[...]


_BN3 = 1000


def kernel(feats, xyz0, sxyz0, sfeats0, xyz1, sxyz1, sfeats1, xyz2, sxyz2, sfeats2, xyz3, sxyz3, sfeats3, u0_ln1_g, u0_ln1_b, u0_w1, u0_b1, u0_ln2_g, u0_ln2_b, u0_w2, u0_b2, u1_ln1_g, u1_ln1_b, u1_w1, u1_b1, u1_ln2_g, u1_ln2_b, u1_w2, u1_b2, u2_ln1_g, u2_ln1_b, u2_w1, u2_b1, u2_ln2_g, u2_ln2_b, u2_w2, u2_b2, u3_ln1_g, u3_ln1_b, u3_w1, u3_b1, u3_ln2_g, u3_ln2_b, u3_w2, u3_b2, conf_w1, conf_b1, conf_bn_g, conf_bn_b, conf_w2, conf_b2):
    r1 = lambda v: v.reshape(1, -1)

    f2_3 = pl.pallas_call(
        _stages012_body,
        out_shape=jax.ShapeDtypeStruct((2500, 128), jnp.float32),
    )(feats, xyz0.T, sxyz0, sfeats0, xyz1.T, sxyz1, sfeats1, xyz2.T, sxyz2,
      sfeats2,
      r1(u0_ln1_g), r1(u0_ln1_b), u0_w1, r1(u0_b1), r1(u0_ln2_g), r1(u0_ln2_b), u0_w2, r1(u0_b2),
      r1(u1_ln1_g), r1(u1_ln1_b), u1_w1, r1(u1_b1), r1(u1_ln2_g), r1(u1_ln2_b), u1_w2, r1(u1_b2),
      r1(u2_ln1_g), r1(u2_ln1_b), u2_w1, r1(u2_b1), r1(u2_ln2_g), r1(u2_ln2_b), u2_w2, r1(u2_b2),
      r1(u3_ln2_g), r1(u3_ln2_b), u3_w2, r1(u3_b2))

    n3 = sxyz3.shape[0]
    grid = (n3 // _BN3,)
    full = lambda s: pl.BlockSpec(s, lambda i: (0,) * len(s))
    blk = lambda s: pl.BlockSpec(s, lambda i: (i,) + (0,) * (len(s) - 1))
    h = pl.pallas_call(
        _stage3_body,
        grid=grid,
        in_specs=[
            blk((_BN3, 3)),
            full((3, 2500)),
            full((2500, 128)),
            blk((_BN3, 128)),
            full((1, 128)), full((1, 128)),
            full((128, 128)), full((1, 128)),
            full((128, 128)), full((1, 128)),
        ],
        out_specs=blk((_BN3, 128)),
        out_shape=jax.ShapeDtypeStruct((n3, 128), jnp.float32),
    )(sxyz3, xyz3.T, f2_3, sfeats3, r1(u3_ln1_g), r1(u3_ln1_b), u3_w1,
      r1(u3_b1), conf_w1, r1(conf_b1))

    conf = pl.pallas_call(
        _conf_body,
        out_shape=jax.ShapeDtypeStruct((n3, 1), jnp.float32),
    )(h, r1(conf_bn_g), r1(conf_bn_b), conf_w2.T, r1(conf_b2))
    return conf


# speedup vs baseline: 7.1779x; 1.6022x over previous
import functools

import jax
import jax.numpy as jnp
from jax.experimental import pallas as pl

_HI = jax.lax.Precision.DEFAULT


def _ln(x, g, b):
    mu = jnp.mean(x, axis=-1, keepdims=True)
    var = jnp.mean((x - mu) ** 2, axis=-1, keepdims=True)
    return (x - mu) / jnp.sqrt(var + 1e-5) * g + b


def _mm(a, b):
    return jax.lax.dot_general(a, b, (((1,), (0,)), ((), ())),
                               precision=_HI, preferred_element_type=jnp.float32)


def _knn_mix(sxyz, xyzT, f2):
    d = None
    for c in range(3):
        diff = sxyz[:, c:c + 1] - xyzT[c:c + 1, :]
        sq = diff * diff
        d = sq if d is None else d + sq
    m_count = xyzT.shape[1]
    iota = jax.lax.broadcasted_iota(jnp.int32, d.shape, 1)
    remaining = d
    sel_w = jnp.zeros_like(d)
    wsum = None
    for r in range(3):
        mval = jnp.min(remaining, axis=1, keepdims=True)
        eq = remaining == mval
        idx = jnp.min(jnp.where(eq, iota, m_count), axis=1, keepdims=True)
        sel = iota == idx
        dist = jnp.sqrt(jnp.maximum(mval, 1e-10))
        w = 1.0 / (dist + 1e-8)
        wsum = w if r == 0 else wsum + w
        sel_w = sel_w + jnp.where(sel, w, 0.0)
        remaining = jnp.where(sel, jnp.inf, remaining)
    return _mm(sel_w / wsum, f2)


def _stages012_body(feats, xyz0T, sxyz0, sfeats0, xyz1T, sxyz1, sfeats1,
                    xyz2T, sxyz2, sfeats2,
                    u0g1, u0b1, u0w1, u0bb1, u0g2, u0b2, u0w2, u0bb2,
                    u1g1, u1b1, u1w1, u1bb1, u1g2, u1b2, u1w2, u1bb2,
                    u2g1, u2b1, u2w1, u2bb1, u2g2, u2b2, u2w2, u2bb2,
                    u3g2, u3b2, u3w2, u3bb2, f2_3_out):
    f = feats[...]
    params = (
        (xyz0T, sxyz0, sfeats0, u0g1, u0b1, u0w1, u0bb1, u0g2, u0b2, u0w2, u0bb2),
        (xyz1T, sxyz1, sfeats1, u1g1, u1b1, u1w1, u1bb1, u1g2, u1b2, u1w2, u1bb2),
        (xyz2T, sxyz2, sfeats2, u2g1, u2b1, u2w1, u2bb1, u2g2, u2b2, u2w2, u2bb2),
    )
    for (xyzT, sxyz, sfeats, g1, b1, w1, bb1, g2, b2, w2, bb2) in params:
        a = _mm(_ln(sfeats[...], g1[...], b1[...]), w1[...]) + bb1[...]
        f2 = _mm(_ln(f, g2[...], b2[...]), w2[...]) + bb2[...]
        f = a + _knn_mix(sxyz[...], xyzT[...], f2)
    f2_3_out[...] = _mm(_ln(f, u3g2[...], u3b2[...]), u3w2[...]) + u3bb2[...]


def _stage3_body(sxyz3, xyz3T, f2_3, sfeats3, g1, b1, w1, bb1,
                 cw1, cb1, h_out):
    a = _mm(_ln(sfeats3[...], g1[...], b1[...]), w1[...]) + bb1[...]
    f = a + _knn_mix(sxyz3[...], xyz3T[...], f2_3[...])
    h_out[...] = _mm(f, cw1[...]) + cb1[...]


def _conf_body(h_ref, bn_g, bn_b, w2T, b2, conf_out):
    h = h_ref[...]
    mu = jnp.mean(h, axis=0, keepdims=True)
    var = jnp.mean((h - mu) ** 2, axis=0, keepdims=True)
    h = (h - mu) / jnp.sqrt(var + 1e-5) * bn_g[...] + bn_b[...]
    h = jnp.maximum(h, 0.0)
    conf_out[...] = jnp.sum(h * w2T[...], axis=1, keepdims=True) + b2[...]


_BN3 = 1000


def kernel(feats, xyz0, sxyz0, sfeats0, xyz1, sxyz1, sfeats1, xyz2, sxyz2, sfeats2, xyz3, sxyz3, sfeats3, u0_ln1_g, u0_ln1_b, u0_w1, u0_b1, u0_ln2_g, u0_ln2_b, u0_w2, u0_b2, u1_ln1_g, u1_ln1_b, u1_w1, u1_b1, u1_ln2_g, u1_ln2_b, u1_w2, u1_b2, u2_ln1_g, u2_ln1_b, u2_w1, u2_b1, u2_ln2_g, u2_ln2_b, u2_w2, u2_b2, u3_ln1_g, u3_ln1_b, u3_w1, u3_b1, u3_ln2_g, u3_ln2_b, u3_w2, u3_b2, conf_w1, conf_b1, conf_bn_g, conf_bn_b, conf_w2, conf_b2):
    r1 = lambda v: v.reshape(1, -1)

    f2_3 = pl.pallas_call(
        _stages012_body,
        out_shape=jax.ShapeDtypeStruct((2500, 128), jnp.float32),
    )(feats, xyz0.T, sxyz0, sfeats0, xyz1.T, sxyz1, sfeats1, xyz2.T, sxyz2,
      sfeats2,
      r1(u0_ln1_g), r1(u0_ln1_b), u0_w1, r1(u0_b1), r1(u0_ln2_g), r1(u0_ln2_b), u0_w2, r1(u0_b2),
      r1(u1_ln1_g), r1(u1_ln1_b), u1_w1, r1(u1_b1), r1(u1_ln2_g), r1(u1_ln2_b), u1_w2, r1(u1_b2),
      r1(u2_ln1_g), r1(u2_ln1_b), u2_w1, r1(u2_b1), r1(u2_ln2_g), r1(u2_ln2_b), u2_w2, r1(u2_b2),
      r1(u3_ln2_g), r1(u3_ln2_b), u3_w2, r1(u3_b2))

    n3 = sxyz3.shape[0]
    grid = (n3 // _BN3,)
    full = lambda s: pl.BlockSpec(s, lambda i: (0,) * len(s))
    blk = lambda s: pl.BlockSpec(s, lambda i: (i,) + (0,) * (len(s) - 1))
    h = pl.pallas_call(
        _stage3_body,
        grid=grid,
        in_specs=[
            blk((_BN3, 3)),
            full((3, 2500)),
            full((2500, 128)),
            blk((_BN3, 128)),
            full((1, 128)), full((1, 128)),
            full((128, 128)), full((1, 128)),
            full((128, 128)), full((1, 128)),
        ],
        out_specs=blk((_BN3, 128)),
        out_shape=jax.ShapeDtypeStruct((n3, 128), jnp.float32),
    )(sxyz3, xyz3.T, f2_3, sfeats3, r1(u3_ln1_g), r1(u3_ln1_b), u3_w1,
      r1(u3_b1), conf_w1, r1(conf_b1))

    conf = pl.pallas_call(
        _conf_body,
        out_shape=jax.ShapeDtypeStruct((n3, 1), jnp.float32),
    )(h, r1(conf_bn_g), r1(conf_bn_b), conf_w2.T, r1(conf_b2))
    return conf
